# baseline (speedup 1.0000x reference)
import functools

import jax
import jax.numpy as jnp
from jax import lax
from jax.experimental import pallas as pl
from jax.experimental.pallas import tpu as pltpu
from jax.experimental.pallas import tpu_sc as plsc

NUM_WORKERS = 32
BBLK = 128
BPAD = BBLK + 1
LANES = 16
NSLOT = 4


def _make_kernel(b: int, l: int, embed: int):
    n_units = l * (b // BBLK)
    per_w = n_units // NUM_WORKERS
    n_groups = per_w // NSLOT
    eblk = embed // 8
    bb_per_l = b // BBLK

    mesh = plsc.VectorSubcoreMesh(core_axis_name="c", subcore_axis_name="s")

    @functools.partial(
        pl.kernel,
        mesh=mesh,
        out_type=jax.ShapeDtypeStruct((n_units * embed, BBLK), jnp.float32),
        scratch_types=[
            pltpu.VMEM((per_w, BBLK), jnp.int32),
            pltpu.VMEM((NSLOT, BBLK, embed), jnp.float32),
            pltpu.VMEM((NSLOT, embed, BPAD), jnp.float32),
            pltpu.SemaphoreType.DMA((NSLOT,)),
            pltpu.SemaphoreType.DMA((NSLOT,)),
        ],
        compiler_params=pltpu.CompilerParams(
            use_tc_tiling_on_sc=False, needs_layout_passes=False
        ),
    )
    def gather_kernel(qlin_hbm, table_hbm, out_hbm, idx_v, rows_v, buf_v,
                      gsem, wsem):
        wid = lax.axis_index("s") * 2 + lax.axis_index("c")
        u0 = wid * per_w

        pltpu.sync_copy(qlin_hbm.at[pl.ds(u0, per_w)], idx_v)

        def gather_start(slot, i):
            pltpu.async_copy(
                table_hbm.at[idx_v.at[i]],
                rows_v.at[slot],
                gsem.at[slot],
            )

        def gather_wait(slot):
            pltpu.make_async_copy(
                table_hbm.at[idx_v.at[0]],
                rows_v.at[slot],
                gsem.at[slot],
            ).wait()

        def write_wait(slot):
            for eb in range(eblk):
                pltpu.make_async_copy(
                    buf_v.at[slot, pl.ds(eb * 8, 8), pl.ds(0, BBLK)],
                    out_hbm.at[pl.ds(0, 8)],
                    wsem.at[slot],
                ).wait()

        e_iotas = [
            lax.iota(jnp.int32, LANES) + k * LANES
            for k in range(embed // LANES)
        ]

        def transpose_unit(slot):
            for bc0 in range(0, BBLK, 8):
                for k in range(embed // LANES):
                    vals = [
                        rows_v[slot, bc0 + j, pl.ds(k * LANES, LANES)]
                        for j in range(8)
                    ]
                    for j in range(8):
                        plsc.store_scatter(
                            buf_v.at[slot],
                            [e_iotas[k], jnp.full((LANES,), bc0 + j, jnp.int32)],
                            vals[j],
                        )

        def write_start(slot, u):
            l_id = u // bb_per_l
            bb = u - l_id * bb_per_l
            for eb in range(eblk):
                base = ((l_id * eblk + eb) * bb_per_l + bb) * 8
                pltpu.async_copy(
                    buf_v.at[slot, pl.ds(eb * 8, 8), pl.ds(0, BBLK)],
                    out_hbm.at[pl.ds(base, 8)],
                    wsem.at[slot],
                )

        for slot in range(NSLOT):
            gather_start(slot, slot)

        def body(g, carry):
            i0 = g * NSLOT
            for slot in range(NSLOT):
                i = i0 + slot
                gather_wait(slot)

                @pl.when(g > 0)
                def _():
                    write_wait(slot)

                transpose_unit(slot)
                write_start(slot, u0 + i)

                @pl.when(g + 1 < n_groups)
                def _():
                    gather_start(slot, i + NSLOT)

            return carry

        lax.fori_loop(0, n_groups, body, 0)
        for slot in range(NSLOT):
            write_wait(slot)

    return gather_kernel


def kernel(q, W):
    b, l = q.shape
    _, embed = W.shape
    qlin = q.T.reshape(l * (b // BBLK), BBLK).astype(jnp.int32)
    out = _make_kernel(b, l, embed)(qlin, W)
    x5 = out.reshape(l, embed // 8, b // BBLK, 8, BBLK)
    return x5.transpose(2, 4, 0, 1, 3).reshape(b, l, embed)


# speedup vs baseline: 1.0009x; 1.0009x over previous
import functools

import jax
import jax.numpy as jnp
from jax import lax
from jax.experimental import pallas as pl
from jax.experimental.pallas import tpu as pltpu
from jax.experimental.pallas import tpu_sc as plsc

NUM_WORKERS = 32
BBLK = 128
BPAD = BBLK + 1
LANES = 16
GU = 2


def _make_kernel(b: int, l: int, embed: int):
    n_units = l * (b // BBLK)
    per_w = n_units // NUM_WORKERS
    n_streams = per_w // GU
    n_groups = n_streams // 2
    eblk = embed // 8
    bb_per_l = b // BBLK

    mesh = plsc.VectorSubcoreMesh(core_axis_name="c", subcore_axis_name="s")

    @functools.partial(
        pl.kernel,
        mesh=mesh,
        out_type=jax.ShapeDtypeStruct((n_units * embed, BBLK), jnp.float32),
        scratch_types=[
            pltpu.VMEM((n_streams, GU * BBLK), jnp.int32),
            pltpu.VMEM((2, GU * BBLK, embed), jnp.float32),
            pltpu.VMEM((2, embed, BPAD), jnp.float32),
            pltpu.SemaphoreType.DMA((2,)),
            pltpu.SemaphoreType.DMA((2,)),
        ],
        compiler_params=pltpu.CompilerParams(
            use_tc_tiling_on_sc=False, needs_layout_passes=False
        ),
    )
    def gather_kernel(qlin_hbm, table_hbm, out_hbm, idx_v, rows_v, buf_v,
                      gsem, wsem):
        wid = lax.axis_index("s") * 2 + lax.axis_index("c")
        u0 = wid * per_w
        s0 = wid * n_streams

        pltpu.sync_copy(qlin_hbm.at[pl.ds(s0, n_streams)], idx_v)

        def gather_start(slot, i):
            pltpu.async_copy(
                table_hbm.at[idx_v.at[i]],
                rows_v.at[slot],
                gsem.at[slot],
            )

        def gather_wait(slot):
            pltpu.make_async_copy(
                table_hbm.at[idx_v.at[0]],
                rows_v.at[slot],
                gsem.at[slot],
            ).wait()

        def write_wait(slot):
            for eb in range(eblk):
                pltpu.make_async_copy(
                    buf_v.at[slot, pl.ds(eb * 8, 8), pl.ds(0, BBLK)],
                    out_hbm.at[pl.ds(0, 8)],
                    wsem.at[slot],
                ).wait()

        e_iotas = [
            lax.iota(jnp.int32, LANES) + k * LANES
            for k in range(embed // LANES)
        ]

        def transpose_unit(slot, h, bslot):
            for bc0 in range(0, BBLK, 8):
                for k in range(embed // LANES):
                    vals = [
                        rows_v[slot, h * BBLK + bc0 + j, pl.ds(k * LANES, LANES)]
                        for j in range(8)
                    ]
                    for j in range(8):
                        plsc.store_scatter(
                            buf_v.at[bslot],
                            [e_iotas[k], jnp.full((LANES,), bc0 + j, jnp.int32)],
                            vals[j],
                        )

        def write_start(slot, u):
            l_id = u // bb_per_l
            bb = u - l_id * bb_per_l
            for eb in range(eblk):
                base = ((l_id * eblk + eb) * bb_per_l + bb) * 8
                pltpu.async_copy(
                    buf_v.at[slot, pl.ds(eb * 8, 8), pl.ds(0, BBLK)],
                    out_hbm.at[pl.ds(base, 8)],
                    wsem.at[slot],
                )

        gather_start(0, 0)
        gather_start(1, 1)

        def body(j, carry):
            for slot in range(2):
                g = j * 2 + slot
                gather_wait(slot)
                for h in range(GU):
                    bslot = h % 2
                    if slot == 0 and h < 2:
                        @pl.when(j > 0)
                        def _():
                            write_wait(bslot)
                    else:
                        write_wait(bslot)
                    transpose_unit(slot, h, bslot)
                    write_start(bslot, u0 + g * GU + h)

                @pl.when(j + 1 < n_groups)
                def _():
                    gather_start(slot, g + 2)

            return carry

        lax.fori_loop(0, n_groups, body, 0)
        write_wait(0)
        write_wait(1)

    return gather_kernel


def kernel(q, W):
    b, l = q.shape
    _, embed = W.shape
    qlin = q.T.reshape(l * (b // BBLK) // GU, GU * BBLK).astype(jnp.int32)
    out = _make_kernel(b, l, embed)(qlin, W)
    x5 = out.reshape(l, embed // 8, b // BBLK, 8, BBLK)
    return x5.transpose(2, 4, 0, 1, 3).reshape(b, l, embed)


# speedup vs baseline: 1.0605x; 1.0595x over previous
import functools

import jax
import jax.numpy as jnp
from jax import lax
from jax.experimental import pallas as pl
from jax.experimental.pallas import tpu as pltpu
from jax.experimental.pallas import tpu_sc as plsc

NUM_WORKERS = 32
BBLK = 128
BPAD = BBLK + 1
LANES = 16


def _make_kernel(b: int, l: int, embed: int):
    n_units = l * (b // BBLK)
    per_w = n_units // NUM_WORKERS
    n_groups = per_w // 2
    eblk = embed // 8
    bb_per_l = b // BBLK

    mesh = plsc.VectorSubcoreMesh(core_axis_name="c", subcore_axis_name="s")

    @functools.partial(
        pl.kernel,
        mesh=mesh,
        out_type=jax.ShapeDtypeStruct((n_units * embed, BBLK), jnp.float32),
        scratch_types=[
            pltpu.VMEM((per_w, BBLK), jnp.int32),
            pltpu.VMEM((2, BBLK, embed), jnp.float32),
            pltpu.VMEM((2, embed, BPAD), jnp.float32),
            pltpu.SemaphoreType.DMA((2,)),
            pltpu.SemaphoreType.DMA((2,)),
        ],
        compiler_params=pltpu.CompilerParams(
            use_tc_tiling_on_sc=False, needs_layout_passes=False
        ),
    )
    def gather_kernel(qlin_hbm, table_hbm, out_hbm, idx_v, rows_v, buf_v,
                      gsem, wsem):
        wid = lax.axis_index("s") * 2 + lax.axis_index("c")
        u0 = wid * per_w

        pltpu.sync_copy(qlin_hbm.at[pl.ds(u0, per_w)], idx_v)

        def gather_start(slot, i):
            pltpu.async_copy(
                table_hbm.at[idx_v.at[i]],
                rows_v.at[slot],
                gsem.at[slot],
            )

        def gather_wait(slot):
            pltpu.make_async_copy(
                table_hbm.at[idx_v.at[0]],
                rows_v.at[slot],
                gsem.at[slot],
            ).wait()

        def write_wait(slot):
            for eb in range(eblk):
                pltpu.make_async_copy(
                    buf_v.at[slot, pl.ds(eb * 8, 8), pl.ds(0, BBLK)],
                    out_hbm.at[pl.ds(0, 8)],
                    wsem.at[slot],
                ).wait()

        e_iotas = [
            lax.iota(jnp.int32, LANES) + k * LANES
            for k in range(embed // LANES)
        ]

        def transpose_unit(slot):
            for bc0 in range(0, BBLK, 8):
                for k in range(embed // LANES):
                    vals = [
                        rows_v[slot, bc0 + j, pl.ds(k * LANES, LANES)]
                        for j in range(8)
                    ]
                    for j in range(8):
                        plsc.store_scatter(
                            buf_v.at[slot],
                            [e_iotas[k], jnp.full((LANES,), bc0 + j, jnp.int32)],
                            vals[j],
                        )

        def write_start(slot, u):
            l_id = u // bb_per_l
            bb = u - l_id * bb_per_l
            for eb in range(eblk):
                base = ((l_id * eblk + eb) * bb_per_l + bb) * 8
                pltpu.async_copy(
                    buf_v.at[slot, pl.ds(eb * 8, 8), pl.ds(0, BBLK)],
                    out_hbm.at[pl.ds(base, 8)],
                    wsem.at[slot],
                )

        gather_start(0, 0)
        gather_start(1, 1)

        def body(g, carry):
            i0 = g * 2
            for slot in range(2):
                i = i0 + slot
                gather_wait(slot)

                @pl.when(g > 0)
                def _():
                    write_wait(slot)

                transpose_unit(slot)
                write_start(slot, u0 + i)

                @pl.when(g + 1 < n_groups)
                def _():
                    gather_start(slot, i + 2)

            return carry

        lax.fori_loop(0, n_groups, body, 0)
        write_wait(0)
        write_wait(1)

    return gather_kernel


def kernel(q, W):
    b, l = q.shape
    _, embed = W.shape
    qlin = q.T.reshape(l * (b // BBLK), BBLK).astype(jnp.int32)
    out = _make_kernel(b, l, embed)(qlin, W)
    x5 = out.reshape(l, embed // 8, b // BBLK, 8, BBLK)
    return x5.transpose(2, 4, 0, 1, 3).reshape(b, l, embed)
